# initial kernel scaffold (unmeasured)
import jax
import jax.numpy as jnp
from jax import lax
from jax.experimental import pallas as pl
from jax.experimental.pallas import tpu as pltpu

N_DEV = 4
B, S, D = 4, 256, 4096
H, Dh, Dr = 32, 128, 64
DC_SH = 128
T = B * S
SCALE = (Dh + Dr) ** -0.5


def _matmul(a, b, bn=None):
    m, k = a.shape
    _, n = b.shape
    bn = bn or min(n, 512)

    def body(a_ref, b_ref, o_ref):
        o_ref[...] = jnp.dot(
            a_ref[...], b_ref[...], preferred_element_type=jnp.float32
        )

    return pl.pallas_call(
        body,
        grid=(n // bn,),
        in_specs=[
            pl.BlockSpec((m, k), lambda j: (0, 0)),
            pl.BlockSpec((k, bn), lambda j: (0, j)),
        ],
        out_specs=pl.BlockSpec((m, bn), lambda j: (0, j)),
        out_shape=jax.ShapeDtypeStruct((m, n), jnp.float32),
    )(a, b)


def _gather_kv(c_sh, wuk_sh, wuv_sh):

    def body(c_ref, uk_ref, uv_ref, k_ref, v_ref,
             c_buf, uk_buf, uv_buf, send_sems, recv_sems):
        my = lax.axis_index("i")
        right = lax.rem(my + 1, N_DEV)
        left = lax.rem(my + N_DEV - 1, N_DEV)

        barrier = pltpu.get_barrier_semaphore()
        for nbr in (left, right):
            pl.semaphore_signal(
                barrier, inc=1,
                device_id=(nbr,), device_id_type=pl.DeviceIdType.MESH,
            )
        pl.semaphore_wait(barrier, 2)

        c_buf[0] = c_ref[...]
        uk_buf[0] = uk_ref[...]
        uv_buf[0] = uv_ref[...]

        for h in range(N_DEV - 1):
            rdmas = []
            for t, buf in enumerate((c_buf, uk_buf, uv_buf)):
                r = pltpu.make_async_remote_copy(
                    src_ref=buf.at[h],
                    dst_ref=buf.at[h + 1],
                    send_sem=send_sems.at[t, h],
                    recv_sem=recv_sems.at[t, h],
                    device_id=(right,),
                    device_id_type=pl.DeviceIdType.MESH,
                )
                r.start()
                rdmas.append(r)
            for r in rdmas:
                r.wait()

        k_ref[...] = jnp.dot(
            c_buf[0], uk_buf[0], preferred_element_type=jnp.float32
        )
        v_ref[...] = jnp.dot(
            c_buf[0], uv_buf[0], preferred_element_type=jnp.float32
        )
        for s in range(1, N_DEV):
            k_ref[...] += jnp.dot(
                c_buf[s], uk_buf[s], preferred_element_type=jnp.float32
            )
            v_ref[...] += jnp.dot(
                c_buf[s], uv_buf[s], preferred_element_type=jnp.float32
            )

    return pl.pallas_call(
        body,
        out_shape=(
            jax.ShapeDtypeStruct((T, D), jnp.float32),
            jax.ShapeDtypeStruct((T, D), jnp.float32),
        ),
        in_specs=[pl.BlockSpec(memory_space=pltpu.VMEM)] * 3,
        out_specs=(
            pl.BlockSpec(memory_space=pltpu.VMEM),
            pl.BlockSpec(memory_space=pltpu.VMEM),
        ),
        scratch_shapes=[
            pltpu.VMEM((N_DEV, T, DC_SH), jnp.float32),
            pltpu.VMEM((N_DEV, DC_SH, D), jnp.float32),
            pltpu.VMEM((N_DEV, DC_SH, D), jnp.float32),
            pltpu.SemaphoreType.DMA((3, N_DEV - 1)),
            pltpu.SemaphoreType.DMA((3, N_DEV - 1)),
        ],
        compiler_params=pltpu.CompilerParams(collective_id=0),
    )(c_sh, wuk_sh, wuv_sh)


def _attention(q4, k4, v4, qr4, kr3):

    def body(q_ref, k_ref, v_ref, qr_ref, kr_ref, o_ref):
        q = q_ref[0, :, 0, :]
        k = k_ref[0, :, 0, :]
        v = v_ref[0, :, 0, :]
        qr = qr_ref[0, :, 0, :]
        kr = kr_ref[0, :, :]
        contract = (((1,), (1,)), ((), ()))
        s = (
            lax.dot_general(q, k, contract, preferred_element_type=jnp.float32)
            + lax.dot_general(qr, kr, contract, preferred_element_type=jnp.float32)
        ) * SCALE
        m = jnp.max(s, axis=-1, keepdims=True)
        p = jnp.exp(s - m)
        p = p / jnp.sum(p, axis=-1, keepdims=True)
        o_ref[0, :, 0, :] = jnp.dot(p, v, preferred_element_type=jnp.float32)

    return pl.pallas_call(
        body,
        grid=(B, H),
        in_specs=[
            pl.BlockSpec((1, S, 1, Dh), lambda b, h: (b, 0, h, 0)),
            pl.BlockSpec((1, S, 1, Dh), lambda b, h: (b, 0, h, 0)),
            pl.BlockSpec((1, S, 1, Dh), lambda b, h: (b, 0, h, 0)),
            pl.BlockSpec((1, S, 1, Dr), lambda b, h: (b, 0, h, 0)),
            pl.BlockSpec((1, S, Dr), lambda b, h: (b, 0, 0)),
        ],
        out_specs=pl.BlockSpec((1, S, 1, Dh), lambda b, h: (b, 0, h, 0)),
        out_shape=jax.ShapeDtypeStruct((B, S, H, Dh), jnp.float32),
    )(q4, k4, v4, qr4, kr3)


def kernel(x, Wdkv, Wuk, Wuv, Wq, Wqr, Wkr, Wo):
    x2 = x.reshape(T, D)
    c = _matmul(x2, Wdkv, bn=DC_SH)
    k_full, v_full = _gather_kv(c, Wuk, Wuv)
    q = _matmul(x2, Wq)
    qr = _matmul(x2, Wqr)
    kr = _matmul(x2, Wkr, bn=Dr)
    o4 = _attention(
        q.reshape(B, S, H, Dh),
        k_full.reshape(B, S, H, Dh),
        v_full.reshape(B, S, H, Dh),
        qr.reshape(B, S, H, Dr),
        kr.reshape(B, S, Dr),
    )
    out = _matmul(o4.reshape(T, D), Wo)
    return out.reshape(B, S, D)


# baseline (device time: 433544 ns/iter reference)
import jax
import jax.numpy as jnp
from jax import lax
from jax.experimental import pallas as pl
from jax.experimental.pallas import tpu as pltpu

N_DEV = 4
B, S, D = 4, 256, 4096
H, Dh, Dr = 32, 128, 64
DC_SH = 128
T = B * S
SCALE = (Dh + Dr) ** -0.5


def _matmul(a, b, bn=None):
    m, k = a.shape
    _, n = b.shape
    bn = bn or min(n, 256)

    def body(a_ref, b_ref, o_ref):
        o_ref[...] = jnp.dot(
            a_ref[...], b_ref[...], preferred_element_type=jnp.float32
        )

    return pl.pallas_call(
        body,
        grid=(n // bn,),
        in_specs=[
            pl.BlockSpec((m, k), lambda j: (0, 0)),
            pl.BlockSpec((k, bn), lambda j: (0, j)),
        ],
        out_specs=pl.BlockSpec((m, bn), lambda j: (0, j)),
        out_shape=jax.ShapeDtypeStruct((m, n), jnp.float32),
    )(a, b)


def _ring_gather(c_sh, wuk_sh, wuv_sh):

    def body(c_ref, uk_ref, uv_ref, c_buf, uk_buf, uv_buf,
             send_sems, recv_sems):
        my = lax.axis_index("i")
        right = lax.rem(my + 1, N_DEV)
        left = lax.rem(my + N_DEV - 1, N_DEV)

        barrier = pltpu.get_barrier_semaphore()
        for nbr in (left, right):
            pl.semaphore_signal(
                barrier, inc=1,
                device_id=(nbr,), device_id_type=pl.DeviceIdType.MESH,
            )
        pl.semaphore_wait(barrier, 2)

        c_buf[0] = c_ref[...]
        uk_buf[0] = uk_ref[...]
        uv_buf[0] = uv_ref[...]

        for h in range(N_DEV - 1):
            rdmas = []
            for t, buf in enumerate((c_buf, uk_buf, uv_buf)):
                r = pltpu.make_async_remote_copy(
                    src_ref=buf.at[h],
                    dst_ref=buf.at[h + 1],
                    send_sem=send_sems.at[t, h],
                    recv_sem=recv_sems.at[t, h],
                    device_id=(right,),
                    device_id_type=pl.DeviceIdType.MESH,
                )
                r.start()
                rdmas.append(r)
            for r in rdmas:
                r.wait()

    return pl.pallas_call(
        body,
        out_shape=(
            jax.ShapeDtypeStruct((N_DEV, T, DC_SH), jnp.float32),
            jax.ShapeDtypeStruct((N_DEV, DC_SH, D), jnp.float32),
            jax.ShapeDtypeStruct((N_DEV, DC_SH, D), jnp.float32),
        ),
        in_specs=[pl.BlockSpec(memory_space=pltpu.VMEM)] * 3,
        out_specs=(
            pl.BlockSpec(memory_space=pltpu.VMEM),
            pl.BlockSpec(memory_space=pltpu.VMEM),
            pl.BlockSpec(memory_space=pltpu.VMEM),
        ),
        scratch_shapes=[
            pltpu.SemaphoreType.DMA((3, N_DEV - 1)),
            pltpu.SemaphoreType.DMA((3, N_DEV - 1)),
        ],
        compiler_params=pltpu.CompilerParams(collective_id=0),
    )(c_sh, wuk_sh, wuv_sh)


def _kv_acc(c_all, uk_all, uv_all, bn=512):

    def body(c_ref, uk_ref, uv_ref, k_ref, v_ref):
        s = pl.program_id(1)

        @pl.when(s == 0)
        def _():
            k_ref[...] = jnp.zeros_like(k_ref)
            v_ref[...] = jnp.zeros_like(v_ref)

        k_ref[...] += jnp.dot(
            c_ref[0], uk_ref[0], preferred_element_type=jnp.float32
        )
        v_ref[...] += jnp.dot(
            c_ref[0], uv_ref[0], preferred_element_type=jnp.float32
        )

    return pl.pallas_call(
        body,
        grid=(D // bn, N_DEV),
        in_specs=[
            pl.BlockSpec((1, T, DC_SH), lambda j, s: (s, 0, 0)),
            pl.BlockSpec((1, DC_SH, bn), lambda j, s: (s, 0, j)),
            pl.BlockSpec((1, DC_SH, bn), lambda j, s: (s, 0, j)),
        ],
        out_specs=(
            pl.BlockSpec((T, bn), lambda j, s: (0, j)),
            pl.BlockSpec((T, bn), lambda j, s: (0, j)),
        ),
        out_shape=(
            jax.ShapeDtypeStruct((T, D), jnp.float32),
            jax.ShapeDtypeStruct((T, D), jnp.float32),
        ),
    )(c_all, uk_all, uv_all)


def _attention(q2, k2, v2, qr2, kr2):

    def body(q_ref, k_ref, v_ref, qr_ref, kr_ref, o_ref):
        kr = kr_ref[...]
        contract = (((1,), (1,)), ((), ()))
        for i in range(2):
            q = q_ref[:, i * Dh:(i + 1) * Dh]
            k = k_ref[:, i * Dh:(i + 1) * Dh]
            v = v_ref[:, i * Dh:(i + 1) * Dh]
            qr = qr_ref[:, i * Dr:(i + 1) * Dr]
            s = (
                lax.dot_general(q, k, contract, preferred_element_type=jnp.float32)
                + lax.dot_general(qr, kr, contract, preferred_element_type=jnp.float32)
            ) * SCALE
            m = jnp.max(s, axis=-1, keepdims=True)
            p = jnp.exp(s - m)
            p = p / jnp.sum(p, axis=-1, keepdims=True)
            o_ref[:, i * Dh:(i + 1) * Dh] = jnp.dot(
                p, v, preferred_element_type=jnp.float32
            )

    return pl.pallas_call(
        body,
        grid=(B, H // 2),
        in_specs=[
            pl.BlockSpec((S, 2 * Dh), lambda b, h: (b, h)),
            pl.BlockSpec((S, 2 * Dh), lambda b, h: (b, h)),
            pl.BlockSpec((S, 2 * Dh), lambda b, h: (b, h)),
            pl.BlockSpec((S, 2 * Dr), lambda b, h: (b, h)),
            pl.BlockSpec((S, Dr), lambda b, h: (b, 0)),
        ],
        out_specs=pl.BlockSpec((S, 2 * Dh), lambda b, h: (b, h)),
        out_shape=jax.ShapeDtypeStruct((T, H * Dh), jnp.float32),
    )(q2, k2, v2, qr2, kr2)


def kernel(x, Wdkv, Wuk, Wuv, Wq, Wqr, Wkr, Wo):
    x2 = x.reshape(T, D)
    c = _matmul(x2, Wdkv, bn=DC_SH)
    c_all, uk_all, uv_all = _ring_gather(c, Wuk, Wuv)
    k_full, v_full = _kv_acc(c_all, uk_all, uv_all)
    q = _matmul(x2, Wq)
    qr = _matmul(x2, Wqr)
    kr = _matmul(x2, Wkr, bn=Dr)
    o2 = _attention(q, k_full, v_full, qr, kr)
    out = _matmul(o2, Wo)
    return out.reshape(B, S, D)


# device time: 333942 ns/iter; 1.2983x vs baseline; 1.2983x over previous
import jax
import jax.numpy as jnp
from jax import lax
from jax.experimental import pallas as pl
from jax.experimental.pallas import tpu as pltpu

N_DEV = 4
B, S, D = 4, 256, 4096
H, Dh, Dr = 32, 128, 64
DC_SH = 128
T = B * S
SCALE = (Dh + Dr) ** -0.5


def _matmul(a, b, bn=None, out_dtype=jnp.float32):
    m, k = a.shape
    _, n = b.shape
    bn = bn or min(n, 256)

    def body(a_ref, b_ref, o_ref):
        o_ref[...] = jnp.dot(
            a_ref[...].astype(jnp.bfloat16),
            b_ref[...].astype(jnp.bfloat16),
            preferred_element_type=jnp.float32,
        ).astype(out_dtype)

    return pl.pallas_call(
        body,
        grid=(n // bn,),
        in_specs=[
            pl.BlockSpec((m, k), lambda j: (0, 0)),
            pl.BlockSpec((k, bn), lambda j: (0, j)),
        ],
        out_specs=pl.BlockSpec((m, bn), lambda j: (0, j)),
        out_shape=jax.ShapeDtypeStruct((m, n), out_dtype),
    )(a, b)


def _ring_gather(c_sh, wuk_sh, wuv_sh):

    def body(c_ref, uk_ref, uv_ref, c_buf, uk_buf, uv_buf,
             send_sems, recv_sems):
        my = lax.axis_index("i")
        right = lax.rem(my + 1, N_DEV)
        left = lax.rem(my + N_DEV - 1, N_DEV)

        barrier = pltpu.get_barrier_semaphore()
        for nbr in (left, right):
            pl.semaphore_signal(
                barrier, inc=1,
                device_id=(nbr,), device_id_type=pl.DeviceIdType.MESH,
            )
        pl.semaphore_wait(barrier, 2)

        c_buf[0] = c_ref[...]
        uk_buf[0] = uk_ref[...]
        uv_buf[0] = uv_ref[...]

        for h in range(N_DEV - 1):
            rdmas = []
            for t, buf in enumerate((c_buf, uk_buf, uv_buf)):
                r = pltpu.make_async_remote_copy(
                    src_ref=buf.at[h],
                    dst_ref=buf.at[h + 1],
                    send_sem=send_sems.at[t, h],
                    recv_sem=recv_sems.at[t, h],
                    device_id=(right,),
                    device_id_type=pl.DeviceIdType.MESH,
                )
                r.start()
                rdmas.append(r)
            for r in rdmas:
                r.wait()

    return pl.pallas_call(
        body,
        out_shape=(
            jax.ShapeDtypeStruct((N_DEV, T, DC_SH), jnp.bfloat16),
            jax.ShapeDtypeStruct((N_DEV, DC_SH, D), jnp.bfloat16),
            jax.ShapeDtypeStruct((N_DEV, DC_SH, D), jnp.bfloat16),
        ),
        in_specs=[pl.BlockSpec(memory_space=pltpu.VMEM)] * 3,
        out_specs=(
            pl.BlockSpec(memory_space=pltpu.VMEM),
            pl.BlockSpec(memory_space=pltpu.VMEM),
            pl.BlockSpec(memory_space=pltpu.VMEM),
        ),
        scratch_shapes=[
            pltpu.SemaphoreType.DMA((3, N_DEV - 1)),
            pltpu.SemaphoreType.DMA((3, N_DEV - 1)),
        ],
        compiler_params=pltpu.CompilerParams(collective_id=0),
    )(c_sh, wuk_sh, wuv_sh)


def _kv_acc(c_all, uk_all, uv_all, bn=512):

    def body(c_ref, uk_ref, uv_ref, k_ref, v_ref):
        s = pl.program_id(1)

        @pl.when(s == 0)
        def _():
            k_ref[...] = jnp.zeros_like(k_ref)
            v_ref[...] = jnp.zeros_like(v_ref)

        k_ref[...] += jnp.dot(
            c_ref[0], uk_ref[0], preferred_element_type=jnp.float32
        )
        v_ref[...] += jnp.dot(
            c_ref[0], uv_ref[0], preferred_element_type=jnp.float32
        )

    return pl.pallas_call(
        body,
        grid=(D // bn, N_DEV),
        in_specs=[
            pl.BlockSpec((1, T, DC_SH), lambda j, s: (s, 0, 0)),
            pl.BlockSpec((1, DC_SH, bn), lambda j, s: (s, 0, j)),
            pl.BlockSpec((1, DC_SH, bn), lambda j, s: (s, 0, j)),
        ],
        out_specs=(
            pl.BlockSpec((T, bn), lambda j, s: (0, j)),
            pl.BlockSpec((T, bn), lambda j, s: (0, j)),
        ),
        out_shape=(
            jax.ShapeDtypeStruct((T, D), jnp.float32),
            jax.ShapeDtypeStruct((T, D), jnp.float32),
        ),
    )(c_all, uk_all, uv_all)


def _attention(q2, k2, v2, qr2, kr2):

    def body(q_ref, k_ref, v_ref, qr_ref, kr_ref, o_ref):
        kr = kr_ref[...].astype(jnp.bfloat16)
        contract = (((1,), (1,)), ((), ()))
        for i in range(2):
            q = q_ref[:, i * Dh:(i + 1) * Dh].astype(jnp.bfloat16)
            k = k_ref[:, i * Dh:(i + 1) * Dh].astype(jnp.bfloat16)
            v = v_ref[:, i * Dh:(i + 1) * Dh].astype(jnp.bfloat16)
            qr = qr_ref[:, i * Dr:(i + 1) * Dr].astype(jnp.bfloat16)
            s = (
                lax.dot_general(q, k, contract, preferred_element_type=jnp.float32)
                + lax.dot_general(qr, kr, contract, preferred_element_type=jnp.float32)
            ) * SCALE
            m = jnp.max(s, axis=-1, keepdims=True)
            p = jnp.exp(s - m)
            p = (p / jnp.sum(p, axis=-1, keepdims=True)).astype(jnp.bfloat16)
            o_ref[:, i * Dh:(i + 1) * Dh] = jnp.dot(
                p, v, preferred_element_type=jnp.float32
            ).astype(jnp.bfloat16)

    return pl.pallas_call(
        body,
        grid=(B, H // 2),
        in_specs=[
            pl.BlockSpec((S, 2 * Dh), lambda b, h: (b, h)),
            pl.BlockSpec((S, 2 * Dh), lambda b, h: (b, h)),
            pl.BlockSpec((S, 2 * Dh), lambda b, h: (b, h)),
            pl.BlockSpec((S, 2 * Dr), lambda b, h: (b, h)),
            pl.BlockSpec((S, Dr), lambda b, h: (b, 0)),
        ],
        out_specs=pl.BlockSpec((S, 2 * Dh), lambda b, h: (b, h)),
        out_shape=jax.ShapeDtypeStruct((T, H * Dh), jnp.bfloat16),
    )(q2, k2, v2, qr2, kr2)


def kernel(x, Wdkv, Wuk, Wuv, Wq, Wqr, Wkr, Wo):
    x2 = x.reshape(T, D)
    bf16 = jnp.bfloat16
    c = _matmul(x2, Wdkv, bn=DC_SH, out_dtype=bf16)
    c_all, uk_all, uv_all = _ring_gather(
        c, Wuk.astype(bf16), Wuv.astype(bf16)
    )
    k_full, v_full = _kv_acc(c_all, uk_all, uv_all)
    q = _matmul(x2, Wq, out_dtype=bf16)
    qr = _matmul(x2, Wqr, out_dtype=bf16)
    kr = _matmul(x2, Wkr, bn=Dr, out_dtype=bf16)
    o2 = _attention(q, k_full, v_full, qr, kr)
    out = _matmul(o2, Wo)
    return out.reshape(B, S, D)


# device time: 304945 ns/iter; 1.4217x vs baseline; 1.0951x over previous
import jax
import jax.numpy as jnp
from jax import lax
from jax.experimental import pallas as pl
from jax.experimental.pallas import tpu as pltpu

N_DEV = 4
B, S, D = 4, 256, 4096
H, Dh, Dr = 32, 128, 64
DC_SH = 128
T = B * S
SCALE = (Dh + Dr) ** -0.5


def _matmul(a, b, bn=None, out_dtype=jnp.float32):
    m, k = a.shape
    _, n = b.shape
    bn = bn or min(n, 256)

    def body(a_ref, b_ref, o_ref):
        o_ref[...] = jnp.dot(
            a_ref[...].astype(jnp.bfloat16),
            b_ref[...].astype(jnp.bfloat16),
            preferred_element_type=jnp.float32,
        ).astype(out_dtype)

    return pl.pallas_call(
        body,
        grid=(n // bn,),
        in_specs=[
            pl.BlockSpec((m, k), lambda j: (0, 0)),
            pl.BlockSpec((k, bn), lambda j: (0, j)),
        ],
        out_specs=pl.BlockSpec((m, bn), lambda j: (0, j)),
        out_shape=jax.ShapeDtypeStruct((m, n), out_dtype),
    )(a, b)


def _ring_gather(c_sh, wuk_sh, wuv_sh):

    def body(c_ref, uk_ref, uv_ref, c_buf, uk_buf, uv_buf,
             send_sems, recv_sems):
        my = lax.axis_index("i")
        right = lax.rem(my + 1, N_DEV)
        left = lax.rem(my + N_DEV - 1, N_DEV)

        barrier = pltpu.get_barrier_semaphore()
        for nbr in (left, right):
            pl.semaphore_signal(
                barrier, inc=1,
                device_id=(nbr,), device_id_type=pl.DeviceIdType.MESH,
            )
        pl.semaphore_wait(barrier, 2)

        c_buf[0] = c_ref[...]
        uk_buf[0] = uk_ref[...]
        uv_buf[0] = uv_ref[...]

        for h in range(N_DEV - 1):
            rdmas = []
            for t, buf in enumerate((c_buf, uk_buf, uv_buf)):
                r = pltpu.make_async_remote_copy(
                    src_ref=buf.at[h],
                    dst_ref=buf.at[h + 1],
                    send_sem=send_sems.at[t, h],
                    recv_sem=recv_sems.at[t, h],
                    device_id=(right,),
                    device_id_type=pl.DeviceIdType.MESH,
                )
                r.start()
                rdmas.append(r)
            for r in rdmas:
                r.wait()

    return pl.pallas_call(
        body,
        out_shape=(
            jax.ShapeDtypeStruct((N_DEV, T, DC_SH), jnp.bfloat16),
            jax.ShapeDtypeStruct((N_DEV, DC_SH, D), jnp.bfloat16),
            jax.ShapeDtypeStruct((N_DEV, DC_SH, D), jnp.bfloat16),
        ),
        in_specs=[pl.BlockSpec(memory_space=pltpu.VMEM)] * 3,
        out_specs=(
            pl.BlockSpec(memory_space=pltpu.VMEM),
            pl.BlockSpec(memory_space=pltpu.VMEM),
            pl.BlockSpec(memory_space=pltpu.VMEM),
        ),
        scratch_shapes=[
            pltpu.SemaphoreType.DMA((3, N_DEV - 1)),
            pltpu.SemaphoreType.DMA((3, N_DEV - 1)),
        ],
        compiler_params=pltpu.CompilerParams(collective_id=0),
    )(c_sh, wuk_sh, wuv_sh)


def _kv_acc(c_all, uk_all, uv_all, bn=512):
    uk2 = uk_all.reshape(N_DEV * DC_SH, D)
    uv2 = uv_all.reshape(N_DEV * DC_SH, D)

    def body(c_ref, uk_ref, uv_ref, k_ref, v_ref):
        c = jnp.concatenate([c_ref[s] for s in range(N_DEV)], axis=1)
        k_ref[...] = jnp.dot(
            c, uk_ref[...], preferred_element_type=jnp.float32
        )
        v_ref[...] = jnp.dot(
            c, uv_ref[...], preferred_element_type=jnp.float32
        )

    return pl.pallas_call(
        body,
        grid=(D // bn,),
        in_specs=[
            pl.BlockSpec((N_DEV, T, DC_SH), lambda j: (0, 0, 0)),
            pl.BlockSpec((N_DEV * DC_SH, bn), lambda j: (0, j)),
            pl.BlockSpec((N_DEV * DC_SH, bn), lambda j: (0, j)),
        ],
        out_specs=(
            pl.BlockSpec((T, bn), lambda j: (0, j)),
            pl.BlockSpec((T, bn), lambda j: (0, j)),
        ),
        out_shape=(
            jax.ShapeDtypeStruct((T, D), jnp.float32),
            jax.ShapeDtypeStruct((T, D), jnp.float32),
        ),
    )(c_all, uk2, uv2)


def _attention(q2, k2, v2, qr2, kr2):

    def body(q_ref, k_ref, v_ref, qr_ref, kr_ref, o_ref):
        kr = kr_ref[...].astype(jnp.bfloat16)
        contract = (((1,), (1,)), ((), ()))
        for i in range(2):
            q = q_ref[:, i * Dh:(i + 1) * Dh].astype(jnp.bfloat16)
            k = k_ref[:, i * Dh:(i + 1) * Dh].astype(jnp.bfloat16)
            v = v_ref[:, i * Dh:(i + 1) * Dh].astype(jnp.bfloat16)
            qr = qr_ref[:, i * Dr:(i + 1) * Dr].astype(jnp.bfloat16)
            s = (
                lax.dot_general(q, k, contract, preferred_element_type=jnp.float32)
                + lax.dot_general(qr, kr, contract, preferred_element_type=jnp.float32)
            ) * SCALE
            p = jnp.exp(s)
            p = (p * (1.0 / jnp.sum(p, axis=-1, keepdims=True))).astype(
                jnp.bfloat16
            )
            o_ref[:, i * Dh:(i + 1) * Dh] = jnp.dot(
                p, v, preferred_element_type=jnp.float32
            ).astype(jnp.bfloat16)

    return pl.pallas_call(
        body,
        grid=(B, H // 2),
        in_specs=[
            pl.BlockSpec((S, 2 * Dh), lambda b, h: (b, h)),
            pl.BlockSpec((S, 2 * Dh), lambda b, h: (b, h)),
            pl.BlockSpec((S, 2 * Dh), lambda b, h: (b, h)),
            pl.BlockSpec((S, 2 * Dr), lambda b, h: (b, h)),
            pl.BlockSpec((S, Dr), lambda b, h: (b, 0)),
        ],
        out_specs=pl.BlockSpec((S, 2 * Dh), lambda b, h: (b, h)),
        out_shape=jax.ShapeDtypeStruct((T, H * Dh), jnp.bfloat16),
    )(q2, k2, v2, qr2, kr2)


def kernel(x, Wdkv, Wuk, Wuv, Wq, Wqr, Wkr, Wo):
    x2 = x.reshape(T, D)
    bf16 = jnp.bfloat16
    c = _matmul(x2, Wdkv, bn=DC_SH, out_dtype=bf16)
    c_all, uk_all, uv_all = _ring_gather(
        c, Wuk.astype(bf16), Wuv.astype(bf16)
    )
    k_full, v_full = _kv_acc(c_all, uk_all, uv_all)
    q = _matmul(x2, Wq, out_dtype=bf16)
    qr = _matmul(x2, Wqr, out_dtype=bf16)
    kr = _matmul(x2, Wkr, bn=Dr, out_dtype=bf16)
    o2 = _attention(q, k_full, v_full, qr, kr)
    out = _matmul(o2, Wo)
    return out.reshape(B, S, D)


# device time: 233861 ns/iter; 1.8539x vs baseline; 1.3040x over previous
import jax
import jax.numpy as jnp
from jax import lax
from jax.experimental import pallas as pl
from jax.experimental.pallas import tpu as pltpu

N_DEV = 4
B, S, D = 4, 256, 4096
H, Dh, Dr = 32, 128, 64
DC_SH = 128
T = B * S
SCALE = (Dh + Dr) ** -0.5


def _matmul(a, b, bn=None, out_dtype=jnp.float32):
    m, k = a.shape
    _, n = b.shape
    bn = bn or min(n, 256)

    def body(a_ref, b_ref, o_ref):
        o_ref[...] = jnp.dot(
            a_ref[...].astype(jnp.bfloat16),
            b_ref[...].astype(jnp.bfloat16),
            preferred_element_type=jnp.float32,
        ).astype(out_dtype)

    return pl.pallas_call(
        body,
        grid=(n // bn,),
        in_specs=[
            pl.BlockSpec((m, k), lambda j: (0, 0)),
            pl.BlockSpec((k, bn), lambda j: (0, j)),
        ],
        out_specs=pl.BlockSpec((m, bn), lambda j: (0, j)),
        out_shape=jax.ShapeDtypeStruct((m, n), out_dtype),
    )(a, b)


def _c_and_cast(x2, wdkv):

    def body(x_ref, w_ref, x16_ref, c_ref):
        xv = x_ref[...].astype(jnp.bfloat16)
        x16_ref[...] = xv
        c_ref[...] = jnp.dot(
            xv, w_ref[...].astype(jnp.bfloat16),
            preferred_element_type=jnp.float32,
        ).astype(jnp.bfloat16)

    return pl.pallas_call(
        body,
        in_specs=[pl.BlockSpec(memory_space=pltpu.VMEM)] * 2,
        out_specs=(
            pl.BlockSpec(memory_space=pltpu.VMEM),
            pl.BlockSpec(memory_space=pltpu.VMEM),
        ),
        out_shape=(
            jax.ShapeDtypeStruct((T, D), jnp.bfloat16),
            jax.ShapeDtypeStruct((T, DC_SH), jnp.bfloat16),
        ),
    )(x2, wdkv)


QBN = 256
QSTEPS = D // QBN


def _gather_q(x16, wq, c16, wuk16, wuv16):
    n_hops = N_DEV - 1

    def body(x_ref, wq_ref, c_ref, uk_ref, uv_ref,
             q_ref, cr_ref, cl_ref, uk_out, uv_out,
             send_sems, recv_sems):
        t = pl.program_id(0)
        my = lax.axis_index("i")
        right = lax.rem(my + 1, N_DEV)
        left = lax.rem(my + N_DEV - 1, N_DEV)

        def hop(h):
            plan = (
                (0, cr_ref, right),
                (1, uk_out, right),
                (2, cl_ref, left),
                (3, uv_out, left),
            )
            return [
                pltpu.make_async_remote_copy(
                    src_ref=buf.at[h],
                    dst_ref=buf.at[h + 1],
                    send_sem=send_sems.at[ti, h],
                    recv_sem=recv_sems.at[ti, h],
                    device_id=(dev,),
                    device_id_type=pl.DeviceIdType.MESH,
                )
                for ti, buf, dev in plan
            ]

        @pl.when(t == 0)
        def _():
            barrier = pltpu.get_barrier_semaphore()
            for nbr in (left, right):
                pl.semaphore_signal(
                    barrier, inc=1,
                    device_id=(nbr,), device_id_type=pl.DeviceIdType.MESH,
                )
            pl.semaphore_wait(barrier, 2)
            cr_ref[0] = c_ref[...]
            cl_ref[0] = c_ref[...]
            uk_out[0] = uk_ref[...]
            uv_out[0] = uv_ref[...]
            for r in hop(0):
                r.start()

        for h in range(1, n_hops):
            @pl.when(t == 6 * h)
            def _(h=h):
                for r in hop(h - 1):
                    r.wait()
                for r in hop(h):
                    r.start()

        @pl.when(t == QSTEPS - 1)
        def _():
            for r in hop(n_hops - 1):
                r.wait()

        q_ref[...] = jnp.dot(
            x_ref[...], wq_ref[...].astype(jnp.bfloat16),
            preferred_element_type=jnp.float32,
        ).astype(jnp.bfloat16)

    return pl.pallas_call(
        body,
        grid=(QSTEPS,),
        in_specs=[
            pl.BlockSpec((T, D), lambda j: (0, 0)),
            pl.BlockSpec((D, QBN), lambda j: (0, j)),
            pl.BlockSpec(memory_space=pltpu.VMEM),
            pl.BlockSpec(memory_space=pltpu.VMEM),
            pl.BlockSpec(memory_space=pltpu.VMEM),
        ],
        out_specs=(
            pl.BlockSpec((T, QBN), lambda j: (0, j)),
            pl.BlockSpec(memory_space=pltpu.VMEM),
            pl.BlockSpec(memory_space=pltpu.VMEM),
            pl.BlockSpec(memory_space=pltpu.VMEM),
            pl.BlockSpec(memory_space=pltpu.VMEM),
        ),
        out_shape=(
            jax.ShapeDtypeStruct((T, D), jnp.bfloat16),
            jax.ShapeDtypeStruct((N_DEV, T, DC_SH), jnp.bfloat16),
            jax.ShapeDtypeStruct((N_DEV, T, DC_SH), jnp.bfloat16),
            jax.ShapeDtypeStruct((N_DEV, DC_SH, D), jnp.bfloat16),
            jax.ShapeDtypeStruct((N_DEV, DC_SH, D), jnp.bfloat16),
        ),
        scratch_shapes=[
            pltpu.SemaphoreType.DMA((4, N_DEV - 1)),
            pltpu.SemaphoreType.DMA((4, N_DEV - 1)),
        ],
        compiler_params=pltpu.CompilerParams(collective_id=0),
    )(x16, wq, c16, wuk16, wuv16)


def _kv_acc(c_r, c_l, uk_all, uv_all, bn=512):
    uk2 = uk_all.reshape(N_DEV * DC_SH, D)
    uv2 = uv_all.reshape(N_DEV * DC_SH, D)

    def body(cr_ref, cl_ref, uk_ref, uv_ref, k_ref, v_ref):
        cr = jnp.concatenate([cr_ref[s] for s in range(N_DEV)], axis=1)
        cl = jnp.concatenate([cl_ref[s] for s in range(N_DEV)], axis=1)
        k_ref[...] = jnp.dot(
            cr, uk_ref[...], preferred_element_type=jnp.float32
        )
        v_ref[...] = jnp.dot(
            cl, uv_ref[...], preferred_element_type=jnp.float32
        )

    return pl.pallas_call(
        body,
        grid=(D // bn,),
        in_specs=[
            pl.BlockSpec((N_DEV, T, DC_SH), lambda j: (0, 0, 0)),
            pl.BlockSpec((N_DEV, T, DC_SH), lambda j: (0, 0, 0)),
            pl.BlockSpec((N_DEV * DC_SH, bn), lambda j: (0, j)),
            pl.BlockSpec((N_DEV * DC_SH, bn), lambda j: (0, j)),
        ],
        out_specs=(
            pl.BlockSpec((T, bn), lambda j: (0, j)),
            pl.BlockSpec((T, bn), lambda j: (0, j)),
        ),
        out_shape=(
            jax.ShapeDtypeStruct((T, D), jnp.float32),
            jax.ShapeDtypeStruct((T, D), jnp.float32),
        ),
    )(c_r, c_l, uk2, uv2)


def _attention(q2, k2, v2, qr2, kr2):

    def body(q_ref, k_ref, v_ref, qr_ref, kr_ref, o_ref):
        kr = kr_ref[...].astype(jnp.bfloat16)
        contract = (((1,), (1,)), ((), ()))
        for i in range(2):
            q = q_ref[:, i * Dh:(i + 1) * Dh].astype(jnp.bfloat16)
            k = k_ref[:, i * Dh:(i + 1) * Dh].astype(jnp.bfloat16)
            v = v_ref[:, i * Dh:(i + 1) * Dh].astype(jnp.bfloat16)
            qr = qr_ref[:, i * Dr:(i + 1) * Dr].astype(jnp.bfloat16)
            s = (
                lax.dot_general(q, k, contract, preferred_element_type=jnp.float32)
                + lax.dot_general(qr, kr, contract, preferred_element_type=jnp.float32)
            ) * SCALE
            p = jnp.exp(s)
            p = (p * (1.0 / jnp.sum(p, axis=-1, keepdims=True))).astype(
                jnp.bfloat16
            )
            o_ref[:, i * Dh:(i + 1) * Dh] = jnp.dot(
                p, v, preferred_element_type=jnp.float32
            ).astype(jnp.bfloat16)

    return pl.pallas_call(
        body,
        grid=(B, H // 2),
        in_specs=[
            pl.BlockSpec((S, 2 * Dh), lambda b, h: (b, h)),
            pl.BlockSpec((S, 2 * Dh), lambda b, h: (b, h)),
            pl.BlockSpec((S, 2 * Dh), lambda b, h: (b, h)),
            pl.BlockSpec((S, 2 * Dr), lambda b, h: (b, h)),
            pl.BlockSpec((S, Dr), lambda b, h: (b, 0)),
        ],
        out_specs=pl.BlockSpec((S, 2 * Dh), lambda b, h: (b, h)),
        out_shape=jax.ShapeDtypeStruct((T, H * Dh), jnp.bfloat16),
    )(q2, k2, v2, qr2, kr2)


def kernel(x, Wdkv, Wuk, Wuv, Wq, Wqr, Wkr, Wo):
    x2 = x.reshape(T, D)
    bf16 = jnp.bfloat16
    x16, c = _c_and_cast(x2, Wdkv)
    q, c_r, c_l, uk_all, uv_all = _gather_q(
        x16, Wq, c, Wuk.astype(bf16), Wuv.astype(bf16)
    )
    k_full, v_full = _kv_acc(c_r, c_l, uk_all, uv_all)
    qr = _matmul(x16, Wqr, out_dtype=bf16)
    kr = _matmul(x16, Wkr, bn=Dr, out_dtype=bf16)
    o2 = _attention(q, k_full, v_full, qr, kr)
    out = _matmul(o2, Wo)
    return out.reshape(B, S, D)
